# baseline (speedup 1.0000x reference)
import functools

import jax
import jax.numpy as jnp
from jax import lax
from jax.experimental import pallas as pl
from jax.experimental.pallas import tpu as pltpu
from jax.experimental.pallas import tpu_sc as plsc

NC = 2
NS = 16
NW = NC * NS


@jax.jit
def _emb_lookup(idx, table):
    B, I, JD = idx.shape
    _, V, D = table.shape
    tiles_per_batch = NW // B
    IS = I // tiles_per_batch
    CH = 16
    n_chunks = IS // CH

    mesh = plsc.VectorSubcoreMesh(core_axis_name="c", subcore_axis_name="s")

    def body(idx_hbm, tab_hbm, out_hbm, idx_v, rows_v, gsem, wsem):
        c = lax.axis_index("c")
        s = lax.axis_index("s")
        wid = s * NC + c
        b = wid // tiles_per_batch
        i0 = lax.rem(wid, tiles_per_batch) * IS

        pltpu.sync_copy(idx_hbm.at[b, pl.ds(i0, IS)], idx_v)

        def do_chunk(p, buf):
            copies = []
            for k in range(CH):
                copies.append(
                    pltpu.async_copy(
                        tab_hbm.at[b].at[idx_v.at[p * CH + k]],
                        rows_v.at[buf, k],
                        gsem,
                    )
                )
            for cp in copies:
                cp.wait()
            pltpu.async_copy(
                rows_v.at[buf],
                out_hbm.at[b, pl.ds(i0 + p * CH, CH)],
                wsem,
            )

        def drain_write(buf):
            pltpu.make_async_copy(
                rows_v.at[buf], out_hbm.at[b, pl.ds(i0, CH)], wsem
            ).wait()

        do_chunk(0, 0)
        do_chunk(1, 1)

        def chunk_body(p, _):
            buf = lax.rem(p, 2)
            drain_write(buf)
            do_chunk(p, buf)
            return ()

        lax.fori_loop(2, n_chunks, chunk_body, (), unroll=False)
        drain_write(0)
        drain_write(1)

    f = pl.kernel(
        body,
        out_type=jax.ShapeDtypeStruct((B, I, JD, D), jnp.float32),
        mesh=mesh,
        scratch_types=[
            pltpu.VMEM((IS, JD), jnp.int32),
            pltpu.VMEM((2, CH, JD, D), jnp.float32),
            pltpu.SemaphoreType.DMA,
            pltpu.SemaphoreType.DMA,
        ],
        compiler_params=pltpu.CompilerParams(use_tc_tiling_on_sc=False),
    )
    return f(idx, table)


def kernel(input, weight):
    return _emb_lookup(input.astype(jnp.int32), weight)


# speedup vs baseline: 1.0314x; 1.0314x over previous
import jax
import jax.numpy as jnp
from jax import lax
from jax.experimental import pallas as pl
from jax.experimental.pallas import tpu as pltpu
from jax.experimental.pallas import tpu_sc as plsc

NC = 2
NS = 16
NW = NC * NS
L = 16


@jax.jit
def _emb_lookup(inp_b, table):
    JD, IT, B, IL = inp_b.shape
    _, V, D = table.shape
    tiles_per_batch = NW // B
    ITS = IT // tiles_per_batch
    DT, DS = D // 8, 8
    TB = DS * IL
    M = JD * ITS

    mesh = plsc.VectorSubcoreMesh(core_axis_name="c", subcore_axis_name="s")

    def body(idx_hbm, tab_hbm, out_hbm, idx_v, rows0, rows1, t0, t1,
             gsem, wsem):
        c = lax.axis_index("c")
        s = lax.axis_index("s")
        wid = s * NC + c
        b = wid // tiles_per_batch
        it0 = lax.rem(wid, tiles_per_batch) * ITS
        rows = (rows0, rows1)
        ts = (t0, t1)

        stage = []
        for j in range(JD):
            stage.append(
                pltpu.async_copy(
                    idx_hbm.at[j, pl.ds(it0, ITS), b], idx_v.at[j], gsem
                )
            )
        for cp in stage:
            cp.wait()

        iota = lax.iota(jnp.int32, L)

        def fire_gather(m, buf):
            j = m // ITS
            itl = lax.rem(m, ITS)
            pltpu.async_copy(
                tab_hbm.at[b].at[idx_v.at[j, itl]], rows[buf], gsem
            )

        def gather_wait(buf):
            pltpu.make_async_copy(
                tab_hbm.at[b, pl.ds(0, IL)], rows[buf], gsem
            ).wait()

        def transpose_block(buf):
            src = rows[buf]
            dst = ts[buf]
            for il in range(IL):
                cid = jnp.full((L,), il, jnp.int32)
                for db in range(D // L):
                    vals = src[il, pl.ds(db * L, L)]
                    plsc.store_scatter(dst, [iota + db * L, cid], vals)

        def fire_writes(m, buf):
            j = m // ITS
            itl = lax.rem(m, ITS)
            for dt in range(DT):
                pltpu.async_copy(
                    ts[buf].at[pl.ds(dt * DS, DS)],
                    out_hbm.at[b, j, dt, it0 + itl],
                    wsem,
                )

        def drain_writes(buf):
            for dt in range(DT):
                pltpu.make_async_copy(
                    ts[buf].at[pl.ds(dt * DS, DS)],
                    out_hbm.at[b, 0, dt, it0],
                    wsem,
                ).wait()

        def step(m, buf, fire_next, drain_prev):
            if fire_next:
                fire_gather(m + 1, 1 - buf)
            gather_wait(buf)
            if drain_prev:
                drain_writes(buf)
            transpose_block(buf)
            fire_writes(m, buf)

        fire_gather(0, 0)
        step(0, 0, True, False)
        step(1, 1, True, False)

        def loop_body(p, _):
            step(2 * p, 0, True, True)
            step(2 * p + 1, 1, True, True)
            return ()

        lax.fori_loop(1, M // 2 - 1, loop_body, (), unroll=False)
        step(M - 2, 0, True, True)
        step(M - 1, 1, False, True)
        drain_writes(0)
        drain_writes(1)

    f = pl.kernel(
        body,
        out_type=jax.ShapeDtypeStruct((B, JD, DT, IT, DS, IL), jnp.float32),
        mesh=mesh,
        scratch_types=[
            pltpu.VMEM((JD, ITS, IL), jnp.int32),
            pltpu.VMEM((IL, D), jnp.float32),
            pltpu.VMEM((IL, D), jnp.float32),
            pltpu.VMEM((D, IL), jnp.float32),
            pltpu.VMEM((D, IL), jnp.float32),
            pltpu.SemaphoreType.DMA,
            pltpu.SemaphoreType.DMA,
        ],
        compiler_params=pltpu.CompilerParams(
            use_tc_tiling_on_sc=False, needs_layout_passes=False
        ),
    )
    return f(inp_b, table)


def kernel(input, weight):
    B, I, JD = input.shape
    Bw, Vw, Dw = weight.shape
    inp_b = (
        input.astype(jnp.int32)
        .reshape(B, I // 128, 128, JD)
        .transpose(3, 1, 0, 2)
    )
    out_b = _emb_lookup(inp_b, weight)
    out = (
        out_b.transpose(0, 3, 5, 1, 2, 4)
        .reshape(B, I, JD, Dw)
    )
    return out


# speedup vs baseline: 1.7173x; 1.6651x over previous
import jax
import jax.numpy as jnp
from jax import lax
from jax.experimental import pallas as pl
from jax.experimental.pallas import tpu as pltpu
from jax.experimental.pallas import tpu_sc as plsc

NC = 2
NS = 16
NW = NC * NS
L = 16


@jax.jit
def _emb_lookup(inp_b, table):
    JD, IT, B, IL = inp_b.shape
    _, V, D = table.shape
    tiles_per_batch = NW // B
    ITS = IT // tiles_per_batch
    DT, DS = D // 8, 8
    TB = DS * IL
    M = JD * ITS

    mesh = plsc.VectorSubcoreMesh(core_axis_name="c", subcore_axis_name="s")

    def body(idx_hbm, tab_hbm, out_hbm, idx_v, rows0, rows1, t0, t1,
             gsem, wsem):
        c = lax.axis_index("c")
        s = lax.axis_index("s")
        wid = s * NC + c
        b = wid // tiles_per_batch
        it0 = lax.rem(wid, tiles_per_batch) * ITS
        rows = (rows0, rows1)
        ts = (t0, t1)

        stage = []
        for j in range(JD):
            stage.append(
                pltpu.async_copy(
                    idx_hbm.at[j, pl.ds(it0, ITS), b], idx_v.at[j], gsem
                )
            )
        for cp in stage:
            cp.wait()

        iota = lax.iota(jnp.int32, L)

        def fire_gather(m, buf):
            j = m // ITS
            itl = lax.rem(m, ITS)
            pltpu.async_copy(
                tab_hbm.at[b].at[idx_v.at[j, itl]], rows[buf], gsem
            )

        def gather_wait(buf):
            pltpu.make_async_copy(
                tab_hbm.at[b, pl.ds(0, IL)], rows[buf], gsem
            ).wait()

        def transpose_block(buf):
            src = rows[buf]
            dst = ts[buf]
            for il in range(IL):
                cid = jnp.full((L,), il, jnp.int32)
                for db in range(D // L):
                    vals = src[il, pl.ds(db * L, L)]
                    plsc.store_scatter(dst, [iota + db * L, cid], vals)


        def fire_writes(m, buf):
            j = m // ITS
            itl = lax.rem(m, ITS)
            for dt in range(DT):
                pltpu.async_copy(
                    ts[buf].at[pl.ds(dt * DS, DS), pl.ds(0, IL)],
                    out_hbm.at[b, j, dt, it0 + itl],
                    wsem,
                )

        def drain_writes(buf):
            for dt in range(DT):
                pltpu.make_async_copy(
                    ts[buf].at[pl.ds(dt * DS, DS), pl.ds(0, IL)],
                    out_hbm.at[b, 0, dt, it0],
                    wsem,
                ).wait()

        def step(m, buf, fire_next, drain_prev):
            if fire_next:
                fire_gather(m + 1, 1 - buf)
            gather_wait(buf)
            if drain_prev:
                drain_writes(buf)
            transpose_block(buf)
            fire_writes(m, buf)

        fire_gather(0, 0)
        step(0, 0, True, False)
        step(1, 1, True, False)

        def loop_body(p, _):
            step(2 * p, 0, True, True)
            step(2 * p + 1, 1, True, True)
            return ()

        lax.fori_loop(1, M // 2 - 1, loop_body, (), unroll=False)
        step(M - 2, 0, True, True)
        step(M - 1, 1, False, True)
        drain_writes(0)
        drain_writes(1)

    f = pl.kernel(
        body,
        out_type=jax.ShapeDtypeStruct((B, JD, DT, IT, DS, IL), jnp.float32),
        mesh=mesh,
        scratch_types=[
            pltpu.VMEM((JD, ITS, IL), jnp.int32),
            pltpu.VMEM((IL, D), jnp.float32),
            pltpu.VMEM((IL, D), jnp.float32),
            pltpu.VMEM((D, IL + 1), jnp.float32),
            pltpu.VMEM((D, IL + 1), jnp.float32),
            pltpu.SemaphoreType.DMA,
            pltpu.SemaphoreType.DMA,
        ],
        compiler_params=pltpu.CompilerParams(
            use_tc_tiling_on_sc=False, needs_layout_passes=False
        ),
    )
    return f(inp_b, table)


def kernel(input, weight):
    B, I, JD = input.shape
    Bw, Vw, Dw = weight.shape
    inp_b = (
        input.astype(jnp.int32)
        .reshape(B, I // 128, 128, JD)
        .transpose(3, 1, 0, 2)
    )
    out_b = _emb_lookup(inp_b, weight)
    out = (
        out_b.transpose(0, 3, 5, 1, 2, 4)
        .reshape(B, I, JD, Dw)
    )
    return out
